# baseline (speedup 1.0000x reference)
import functools

import jax
import jax.numpy as jnp
from jax import lax
from jax.experimental import pallas as pl
from jax.experimental.pallas import tpu as pltpu
from jax.experimental.pallas import tpu_sc as plsc

F_ITEMS = 64
L = 512
H = 768
NIDX = 48
NTASK = 3 * F_ITEMS
HV = H // 16

_info = plsc.get_sparse_core_info()
NC, NS = _info.num_cores, _info.num_subcores
NW = NC * NS
TPW = NTASK // NW


def _sc_body(table_hbm, idx_hbm, out_hbm, idx_v, rows_v, acc_v, sem):
    wid = lax.axis_index("s") * NC + lax.axis_index("c")
    scale = jnp.float32(1.0 / NIDX)
    for k in range(TPW):
        t = wid * TPW + k
        item = lax.rem(t, F_ITEMS)
        base = (item * L).astype(jnp.int32)
        pltpu.sync_copy(idx_hbm.at[t], idx_v)
        for j in range(NIDX // 16):
            sl = pl.ds(j * 16, 16)
            idx_v[sl] = idx_v[sl] + base
        pltpu.async_copy(table_hbm.at[idx_v], rows_v, sem).wait()
        for h in range(HV):
            sl = pl.ds(h * 16, 16)
            acc_v[sl] = rows_v[0, sl] + rows_v[1, sl]

        def rbody(r, _):
            for h in range(HV):
                sl = pl.ds(h * 16, 16)
                plsc.addupdate(acc_v.at[sl], rows_v[r, sl])
            return 0

        lax.fori_loop(2, NIDX, rbody, 0)
        for h in range(HV):
            sl = pl.ds(h * 16, 16)
            acc_v[sl] = acc_v[sl] * scale
        pltpu.sync_copy(acc_v, out_hbm.at[t])


@jax.jit
def _pooled(table, idx_all):
    mesh = plsc.VectorSubcoreMesh(core_axis_name="c", subcore_axis_name="s")
    return pl.kernel(
        _sc_body,
        out_type=jax.ShapeDtypeStruct((NTASK, H), jnp.float32),
        mesh=mesh,
        scratch_types=[
            pltpu.VMEM((NIDX,), jnp.int32),
            pltpu.VMEM((NIDX, H), jnp.float32),
            pltpu.VMEM((H,), jnp.float32),
            pltpu.SemaphoreType.DMA,
        ],
    )(table, idx_all)


def kernel(token_emb, idx_V, idx_A0, idx_A1, B, N_max):
    Fdim, Ldim, Hdim = token_emb.shape
    table = token_emb.reshape(Fdim * Ldim, Hdim)
    idx_all = jnp.concatenate(
        [idx_V.reshape(Fdim, NIDX),
         idx_A0.reshape(Fdim, NIDX),
         idx_A1.reshape(Fdim, NIDX)], axis=0)
    out = _pooled(table, idx_all)
    return (out[:Fdim], out[Fdim:2 * Fdim], out[2 * Fdim:])


# speedup vs baseline: 1.1320x; 1.1320x over previous
import functools

import jax
import jax.numpy as jnp
from jax import lax
from jax.experimental import pallas as pl
from jax.experimental.pallas import tpu as pltpu
from jax.experimental.pallas import tpu_sc as plsc

F_ITEMS = 64
L = 512
H = 768
NIDX = 48
NTASK = 3 * F_ITEMS
HV = H // 16

_info = plsc.get_sparse_core_info()
NC, NS = _info.num_cores, _info.num_subcores
NW = NC * NS
TPW = NTASK // NW


def _sc_body(table_hbm, idx_hbm, out_hbm, idx_v, rows_v, acc_v, sems):
    wid = lax.axis_index("s") * NC + lax.axis_index("c")
    scale = jnp.float32(1.0 / NIDX)
    t0 = wid * TPW
    pltpu.sync_copy(idx_hbm.at[wid], idx_v)
    for k in range(TPW):
        base = (lax.rem(t0 + k, F_ITEMS) * L).astype(jnp.int32)
        for j in range(NIDX // 16):
            sl = pl.ds(j * 16, 16)
            idx_v[k, sl] = idx_v[k, sl] + base

    def start(k):
        pltpu.async_copy(table_hbm.at[idx_v.at[k]], rows_v.at[k % 2],
                         sems.at[k % 2])

    start(0)
    for k in range(TPW):
        if k + 1 < TPW:
            start(k + 1)
        pltpu.make_async_copy(table_hbm.at[idx_v.at[k]], rows_v.at[k % 2],
                              sems.at[k % 2]).wait()
        rv = rows_v.at[k % 2]
        for h in range(HV):
            sl = pl.ds(h * 16, 16)
            acc_v[k, sl] = rv[0, sl] + rv[1, sl]

        def rbody(r, _):
            for h in range(HV):
                sl = pl.ds(h * 16, 16)
                plsc.addupdate(acc_v.at[k, sl], rv[r, sl])
            return 0

        lax.fori_loop(2, NIDX, rbody, 0)
        for h in range(HV):
            sl = pl.ds(h * 16, 16)
            acc_v[k, sl] = acc_v[k, sl] * scale
    pltpu.sync_copy(acc_v, out_hbm.at[wid])


@jax.jit
def _pooled(table, idx_all):
    mesh = plsc.VectorSubcoreMesh(core_axis_name="c", subcore_axis_name="s")
    return pl.kernel(
        _sc_body,
        out_type=jax.ShapeDtypeStruct((NW, TPW, H), jnp.float32),
        mesh=mesh,
        scratch_types=[
            pltpu.VMEM((TPW, NIDX), jnp.int32),
            pltpu.VMEM((2, NIDX, H), jnp.float32),
            pltpu.VMEM((TPW, H), jnp.float32),
            pltpu.SemaphoreType.DMA((2,)),
        ],
    )(table, idx_all)


def kernel(token_emb, idx_V, idx_A0, idx_A1, B, N_max):
    Fdim, Ldim, Hdim = token_emb.shape
    table = token_emb.reshape(Fdim * Ldim, Hdim)
    idx_all = jnp.concatenate(
        [idx_V.reshape(Fdim, NIDX),
         idx_A0.reshape(Fdim, NIDX),
         idx_A1.reshape(Fdim, NIDX)], axis=0).reshape(NW, TPW, NIDX)
    out = _pooled(table, idx_all).reshape(NTASK, H)
    return (out[:Fdim], out[Fdim:2 * Fdim], out[2 * Fdim:])


# speedup vs baseline: 2.3554x; 2.0808x over previous
import functools

import jax
import jax.numpy as jnp
from jax import lax
from jax.experimental import pallas as pl
from jax.experimental.pallas import tpu as pltpu
from jax.experimental.pallas import tpu_sc as plsc

F_ITEMS = 64
L = 512
H = 768
NIDX = 48
NTASK = 3 * F_ITEMS
HV = H // 16

_info = plsc.get_sparse_core_info()
NC, NS = _info.num_cores, _info.num_subcores
NW = NC * NS
TPW = NTASK // NW


def _sc_body(table_hbm, idx_hbm, out_hbm, idx_v, rows_v, acc_v, sems):
    wid = lax.axis_index("s") * NC + lax.axis_index("c")
    scale = jnp.float32(1.0 / NIDX)
    t0 = wid * TPW
    pltpu.sync_copy(idx_hbm.at[wid], idx_v)
    for k in range(TPW):
        base = (lax.rem(t0 + k, F_ITEMS) * L).astype(jnp.int32)
        for j in range(NIDX // 16):
            sl = pl.ds(j * 16, 16)
            idx_v[k, sl] = idx_v[k, sl] + base

    def start(k):
        pltpu.async_copy(table_hbm.at[idx_v.at[k]], rows_v.at[k % 2],
                         sems.at[k % 2])

    start(0)
    for k in range(TPW):
        if k + 1 < TPW:
            start(k + 1)
        pltpu.make_async_copy(table_hbm.at[idx_v.at[k]], rows_v.at[k % 2],
                              sems.at[k % 2]).wait()
        rv = rows_v.at[k % 2]

        def hbody(h, _):
            sl = pl.ds(h * 16, 16)
            a = [rv[r, sl] for r in range(4)]
            for r in range(4, NIDX, 4):
                for j in range(4):
                    a[j] = a[j] + rv[r + j, sl]
            acc_v[k, sl] = ((a[0] + a[1]) + (a[2] + a[3])) * scale
            return 0

        lax.fori_loop(0, HV, hbody, 0)
    pltpu.sync_copy(acc_v, out_hbm.at[wid])


@jax.jit
def _pooled(table, idx_all):
    mesh = plsc.VectorSubcoreMesh(core_axis_name="c", subcore_axis_name="s")
    return pl.kernel(
        _sc_body,
        out_type=jax.ShapeDtypeStruct((NW, TPW, H), jnp.float32),
        mesh=mesh,
        scratch_types=[
            pltpu.VMEM((TPW, NIDX), jnp.int32),
            pltpu.VMEM((2, NIDX, H), jnp.float32),
            pltpu.VMEM((TPW, H), jnp.float32),
            pltpu.SemaphoreType.DMA((2,)),
        ],
    )(table, idx_all)


def kernel(token_emb, idx_V, idx_A0, idx_A1, B, N_max):
    Fdim, Ldim, Hdim = token_emb.shape
    table = token_emb.reshape(Fdim * Ldim, Hdim)
    idx_all = jnp.concatenate(
        [idx_V.reshape(Fdim, NIDX),
         idx_A0.reshape(Fdim, NIDX),
         idx_A1.reshape(Fdim, NIDX)], axis=0).reshape(NW, TPW, NIDX)
    out = _pooled(table, idx_all).reshape(NTASK, H)
    return (out[:Fdim], out[Fdim:2 * Fdim], out[2 * Fdim:])


# speedup vs baseline: 2.4459x; 1.0384x over previous
import functools

import jax
import jax.numpy as jnp
from jax import lax
from jax.experimental import pallas as pl
from jax.experimental.pallas import tpu as pltpu
from jax.experimental.pallas import tpu_sc as plsc

F_ITEMS = 64
L = 512
H = 768
P = 8
S = 6
NIDX = P * S
HV = H // 16

_info = plsc.get_sparse_core_info()
NC, NS = _info.num_cores, _info.num_subcores
NW = NC * NS
IPW = F_ITEMS // NW
NT = 3 * IPW


def _sc_body(emb, iv, ia0, ia1, ov, oa0, oa1, idx_v, rows_v, acc_v,
             gsem, osem, isem):
    wid = lax.axis_index("s") * NC + lax.axis_index("c")
    f0 = wid * IPW
    scale = jnp.float32(1.0 / NIDX)
    idx_refs = (iv, ia0, ia1)
    out_refs = (ov, oa0, oa1)
    tasks = [(a, j) for a in range(3) for j in range(IPW)]

    for t, (a, j) in enumerate(tasks):
        pltpu.async_copy(idx_refs[a].at[f0 + j], idx_v.at[t], isem)
    for t, (a, j) in enumerate(tasks):
        pltpu.make_async_copy(idx_refs[a].at[f0 + j], idx_v.at[t], isem).wait()

    def start(t, a, j):
        pltpu.async_copy(emb.at[f0 + j].at[idx_v.at[t]], rows_v.at[t % 2],
                         gsem.at[t % 2])

    start(0, *tasks[0])
    for t, (a, j) in enumerate(tasks):
        if t + 1 < NT:
            start(t + 1, *tasks[t + 1])
        pltpu.make_async_copy(emb.at[f0 + j].at[idx_v.at[t]],
                              rows_v.at[t % 2], gsem.at[t % 2]).wait()
        rv = rows_v.at[t % 2]

        def hbody(h, _):
            sl = pl.ds(h * 16, 16)
            a4 = [rv[r, sl] for r in range(4)]
            for r in range(4, NIDX, 4):
                for q in range(4):
                    a4[q] = a4[q] + rv[r + q, sl]
            acc_v[t, sl] = ((a4[0] + a4[1]) + (a4[2] + a4[3])) * scale
            return 0

        lax.fori_loop(0, HV, hbody, 0)
        pltpu.async_copy(acc_v.at[t], out_refs[a].at[f0 + j], osem)
    for t, (a, j) in enumerate(tasks):
        pltpu.make_async_copy(acc_v.at[t], out_refs[a].at[f0 + j], osem).wait()


@jax.jit
def _pooled(emb, iv, ia0, ia1):
    mesh = plsc.VectorSubcoreMesh(core_axis_name="c", subcore_axis_name="s")
    row = jax.ShapeDtypeStruct((F_ITEMS, H), jnp.float32)
    return pl.kernel(
        _sc_body,
        out_type=(row, row, row),
        mesh=mesh,
        scratch_types=[
            pltpu.VMEM((NT, NIDX), jnp.int32),
            pltpu.VMEM((2, NIDX, H), jnp.float32),
            pltpu.VMEM((NT, H), jnp.float32),
            pltpu.SemaphoreType.DMA((2,)),
            pltpu.SemaphoreType.DMA,
            pltpu.SemaphoreType.DMA,
        ],
    )(emb, iv, ia0, ia1)


def kernel(token_emb, idx_V, idx_A0, idx_A1, B, N_max):
    Fdim = token_emb.shape[0]
    e_V, e_A0, e_A1 = _pooled(token_emb,
                              idx_V.reshape(Fdim, NIDX),
                              idx_A0.reshape(Fdim, NIDX),
                              idx_A1.reshape(Fdim, NIDX))
    return (e_V, e_A0, e_A1)


# speedup vs baseline: 2.4787x; 1.0134x over previous
import functools

import jax
import jax.numpy as jnp
from jax import lax
from jax.experimental import pallas as pl
from jax.experimental.pallas import tpu as pltpu
from jax.experimental.pallas import tpu_sc as plsc

F_ITEMS = 64
L = 512
H = 768
P = 8
S = 6
NIDX = P * S
HV = H // 16

_info = plsc.get_sparse_core_info()
NC, NS = _info.num_cores, _info.num_subcores
NW = NC * NS
IPW = F_ITEMS // NW
NT = 3 * IPW


def _sc_body(emb, iv, ia0, ia1, ov, oa0, oa1, idx_v, rows_v, acc_v,
             gsem, osem, isem):
    wid = lax.axis_index("s") * NC + lax.axis_index("c")
    f0 = wid * IPW
    scale = jnp.float32(1.0 / NIDX)
    idx_refs = (iv, ia0, ia1)
    out_refs = (ov, oa0, oa1)
    tasks = [(a, j) for a in range(3) for j in range(IPW)]

    for t, (a, j) in enumerate(tasks):
        pltpu.async_copy(idx_refs[a].at[f0 + j], idx_v.at[t], isem)
    for t, (a, j) in enumerate(tasks):
        pltpu.make_async_copy(idx_refs[a].at[f0 + j], idx_v.at[t],
                              isem).wait()

    def start(t, a, j):
        pltpu.async_copy(emb.at[f0 + j].at[idx_v.at[t]], rows_v.at[t % 3],
                         gsem.at[t % 3])

    start(0, *tasks[0])
    start(1, *tasks[1])
    for t, (a, j) in enumerate(tasks):
        if t + 2 < NT:
            start(t + 2, *tasks[t + 2])
        pltpu.make_async_copy(emb.at[f0 + j].at[idx_v.at[t]],
                              rows_v.at[t % 3], gsem.at[t % 3]).wait()
        rv = rows_v.at[t % 3]

        def hbody(h, _):
            sl = pl.ds(h * 16, 16)
            a4 = [rv[r, sl] for r in range(4)]
            for r in range(4, NIDX, 4):
                for q in range(4):
                    a4[q] = a4[q] + rv[r + q, sl]
            acc_v[t, sl] = ((a4[0] + a4[1]) + (a4[2] + a4[3])) * scale
            return 0

        lax.fori_loop(0, HV, hbody, 0)
        pltpu.async_copy(acc_v.at[t], out_refs[a].at[f0 + j], osem)
    for t, (a, j) in enumerate(tasks):
        pltpu.make_async_copy(acc_v.at[t], out_refs[a].at[f0 + j], osem).wait()


@jax.jit
def _pooled(emb, iv, ia0, ia1):
    mesh = plsc.VectorSubcoreMesh(core_axis_name="c", subcore_axis_name="s")
    row = jax.ShapeDtypeStruct((F_ITEMS, H), jnp.float32)
    return pl.kernel(
        _sc_body,
        out_type=(row, row, row),
        mesh=mesh,
        scratch_types=[
            pltpu.VMEM((NT, NIDX), jnp.int32),
            pltpu.VMEM((3, NIDX, H), jnp.float32),
            pltpu.VMEM((NT, H), jnp.float32),
            pltpu.SemaphoreType.DMA((3,)),
            pltpu.SemaphoreType.DMA,
            pltpu.SemaphoreType.DMA,
        ],
    )(emb, iv, ia0, ia1)


def kernel(token_emb, idx_V, idx_A0, idx_A1, B, N_max):
    Fdim = token_emb.shape[0]
    return _pooled(token_emb,
                   idx_V.reshape(Fdim, NIDX),
                   idx_A0.reshape(Fdim, NIDX),
                   idx_A1.reshape(Fdim, NIDX))
